# baseline (speedup 1.0000x reference)
import functools

import jax
import jax.numpy as jnp
from jax import lax
from jax.experimental import pallas as pl
from jax.experimental.pallas import tpu as pltpu
from jax.experimental.pallas import tpu_sc as plsc

N = 10000
NPAD = 10240
D = 128
E = 320000
ET = E + N
NC = 2
NS = 16
CHUNK = 128
CPT = 80
CAP = NC * NS * CPT * CHUNK
ROWS_PER_TILE_PAD = NPAD // NS
ROWS_PER_TILE_OUT = N // NS
ALPHA = 0.1
K = 10
RB = 2000

_f32 = jnp.float32


def _sc_mesh():
    return plsc.VectorSubcoreMesh(core_axis_name="c", subcore_axis_name="s")


def _make_propagate():
    @functools.partial(
        pl.kernel,
        mesh=_sc_mesh(),
        out_type=jax.ShapeDtypeStruct((NC, NPAD, D), _f32),
        scratch_types=[
            pltpu.VMEM((CHUNK,), jnp.int32),
            pltpu.VMEM((CHUNK,), jnp.int32),
            pltpu.VMEM((CHUNK, D), _f32),
            pltpu.VMEM_SHARED((NPAD, D), _f32),
            pltpu.SemaphoreType.DMA,
        ],
    )
    def propagate(g_hbm, rows_hbm, cols_hbm, zeros_hbm, acc_hbm,
                  idx_v, cidx_v, buf_v, acc_sh, sem):
        cid = lax.axis_index("c")
        sid = lax.axis_index("s")
        pltpu.sync_copy(zeros_hbm,
                        acc_sh.at[pl.ds(sid * ROWS_PER_TILE_PAD,
                                        ROWS_PER_TILE_PAD)])
        plsc.subcore_barrier()

        def body(c, carry):
            pltpu.sync_copy(rows_hbm.at[cid, sid, c], idx_v)
            pltpu.async_copy(g_hbm.at[idx_v], buf_v, sem).wait()
            pltpu.sync_copy(cols_hbm.at[cid, sid, c], cidx_v)
            pltpu.sync_copy(buf_v, acc_sh.at[cidx_v], add=True)
            return carry

        lax.fori_loop(0, CPT, body, 0)
        plsc.subcore_barrier()
        pltpu.sync_copy(
            acc_sh.at[pl.ds(sid * ROWS_PER_TILE_PAD, ROWS_PER_TILE_PAD)],
            acc_hbm.at[cid, pl.ds(sid * ROWS_PER_TILE_PAD,
                                  ROWS_PER_TILE_PAD)])

    return propagate


def _make_degree():
    @functools.partial(
        pl.kernel,
        mesh=_sc_mesh(),
        out_type=jax.ShapeDtypeStruct((NC, NPAD, 16), _f32),
        scratch_types=[
            pltpu.VMEM((CHUNK,), jnp.int32),
            pltpu.VMEM((CHUNK, 16), _f32),
            pltpu.VMEM_SHARED((NPAD, 16), _f32),
        ],
    )
    def degree(cols_hbm, ones_hbm, zeros_hbm, deg_hbm,
               cidx_v, ones_v, deg_sh):
        cid = lax.axis_index("c")
        sid = lax.axis_index("s")
        pltpu.sync_copy(ones_hbm, ones_v)
        pltpu.sync_copy(zeros_hbm,
                        deg_sh.at[pl.ds(sid * ROWS_PER_TILE_PAD,
                                        ROWS_PER_TILE_PAD)])
        plsc.subcore_barrier()

        def body(c, carry):
            pltpu.sync_copy(cols_hbm.at[cid, sid, c], cidx_v)
            pltpu.sync_copy(ones_v, deg_sh.at[cidx_v], add=True)
            return carry

        lax.fori_loop(0, CPT, body, 0)
        plsc.subcore_barrier()
        pltpu.sync_copy(
            deg_sh.at[pl.ds(sid * ROWS_PER_TILE_PAD, ROWS_PER_TILE_PAD)],
            deg_hbm.at[cid, pl.ds(sid * ROWS_PER_TILE_PAD,
                                  ROWS_PER_TILE_PAD)])

    return degree


def _mlp_call(x, w1t, b1r, w2t, b2r, degp):
    def body(x_ref, w1_ref, b1_ref, w2_ref, b2_ref, dg_ref, h0_ref, g0_ref):
        h1 = jnp.dot(x_ref[...], w1_ref[...],
                     preferred_element_type=_f32) + b1_ref[...]
        h1 = jnp.maximum(h1, 0.0)
        h2 = jnp.dot(h1, w2_ref[...],
                     preferred_element_type=_f32) + b2_ref[...]
        deg = dg_ref[0, :, 0:1] + dg_ref[1, :, 0:1] + 1.0
        h0_ref[...] = h2
        g0_ref[...] = h2 * lax.rsqrt(deg)

    return pl.pallas_call(
        body,
        grid=(N // RB,),
        in_specs=[
            pl.BlockSpec((RB, D), lambda i: (i, 0)),
            pl.BlockSpec((D, D), lambda i: (0, 0)),
            pl.BlockSpec((1, D), lambda i: (0, 0)),
            pl.BlockSpec((D, D), lambda i: (0, 0)),
            pl.BlockSpec((1, D), lambda i: (0, 0)),
            pl.BlockSpec((NC, RB, 16), lambda i: (0, i, 0)),
        ],
        out_specs=[pl.BlockSpec((RB, D), lambda i: (i, 0)),
                   pl.BlockSpec((RB, D), lambda i: (i, 0))],
        out_shape=[jax.ShapeDtypeStruct((N, D), _f32),
                   jax.ShapeDtypeStruct((N, D), _f32)],
    )(x, w1t, b1r, w2t, b2r, degp)


def _update_call(acc, degp, g0, g):
    first = g is g0

    def body(acc_ref, dg_ref, g0_ref, *rest):
        gin_ref = g0_ref if first else rest[0]
        g_ref = rest[-1]
        deg = dg_ref[0, :, 0:1] + dg_ref[1, :, 0:1] + 1.0
        s = acc_ref[0] + acc_ref[1] + gin_ref[...]
        g_ref[...] = ((1.0 - ALPHA) / deg) * s + ALPHA * g0_ref[...]

    in_specs = [
        pl.BlockSpec((NC, RB, D), lambda i: (0, i, 0)),
        pl.BlockSpec((NC, RB, 16), lambda i: (0, i, 0)),
        pl.BlockSpec((RB, D), lambda i: (i, 0)),
    ]
    args = (acc, degp, g0)
    if not first:
        in_specs.append(pl.BlockSpec((RB, D), lambda i: (i, 0)))
        args = args + (g,)
    return pl.pallas_call(
        body,
        grid=(N // RB,),
        in_specs=in_specs,
        out_specs=pl.BlockSpec((RB, D), lambda i: (i, 0)),
        out_shape=jax.ShapeDtypeStruct((N, D), _f32),
    )(*args)


def _final_call(acc, degp, h0, g):
    def body(acc_ref, dg_ref, h0_ref, gin_ref, o_ref):
        deg = dg_ref[0, :, 0:1] + dg_ref[1, :, 0:1] + 1.0
        s = acc_ref[0] + acc_ref[1] + gin_ref[...]
        h = (1.0 - ALPHA) * lax.rsqrt(deg) * s + ALPHA * h0_ref[...]
        m = jnp.max(h, axis=1, keepdims=True)
        ex = jnp.exp(h - m)
        o_ref[...] = h - m - jnp.log(jnp.sum(ex, axis=1, keepdims=True))

    return pl.pallas_call(
        body,
        grid=(N // RB,),
        in_specs=[
            pl.BlockSpec((NC, RB, D), lambda i: (0, i, 0)),
            pl.BlockSpec((NC, RB, 16), lambda i: (0, i, 0)),
            pl.BlockSpec((RB, D), lambda i: (i, 0)),
            pl.BlockSpec((RB, D), lambda i: (i, 0)),
        ],
        out_specs=pl.BlockSpec((RB, D), lambda i: (i, 0)),
        out_shape=jax.ShapeDtypeStruct((N, D), _f32),
    )(acc, degp, h0, g)


def kernel(x, edge_index, W1, b1, W2, b2):
    row = edge_index[0]
    col = edge_index[1]
    rows = jnp.concatenate([row, jnp.zeros((CAP - E,), jnp.int32)])
    cols = jnp.concatenate([col, jnp.full((CAP - E,), N, jnp.int32)])
    rows_a = rows.reshape(NC, NS, CPT, CHUNK)
    cols_a = cols.reshape(NC, NS, CPT, CHUNK)

    zeros_d = jnp.zeros((ROWS_PER_TILE_PAD, D), _f32)
    zeros16 = jnp.zeros((ROWS_PER_TILE_PAD, 16), _f32)
    ones16 = jnp.ones((CHUNK, 16), _f32)

    degree = _make_degree()
    propagate = _make_propagate()

    degp = degree(cols_a, ones16, zeros16)
    h0, g0 = _mlp_call(x, W1.T, b1.reshape(1, D), W2.T, b2.reshape(1, D),
                       degp)

    g = g0
    for _ in range(K - 1):
        acc = propagate(g, rows_a, cols_a, zeros_d)
        g = _update_call(acc, degp, g0, g)
    acc = propagate(g, rows_a, cols_a, zeros_d)
    return _final_call(acc, degp, h0, g)


# speedup vs baseline: 2.0606x; 2.0606x over previous
import functools

import jax
import jax.numpy as jnp
from jax import lax
from jax.experimental import pallas as pl
from jax.experimental.pallas import tpu as pltpu
from jax.experimental.pallas import tpu_sc as plsc

N = 10000
NPAD = 10240
D = 128
E = 320000
ET = E + N
NC = 2
NS = 16
CHUNK = 128
CPT = 81
CAP = NC * NS * CPT * CHUNK
ROWS_PER_TILE_PAD = NPAD // NS
ALPHA = 0.1
K = 10
RB = 2000

_f32 = jnp.float32


def _sc_mesh():
    return plsc.VectorSubcoreMesh(core_axis_name="c", subcore_axis_name="s")


def _make_propagate():
    @functools.partial(
        pl.kernel,
        mesh=_sc_mesh(),
        out_type=jax.ShapeDtypeStruct((NC, NPAD, D), _f32),
        scratch_types=[
            pltpu.VMEM((CHUNK,), jnp.int32),
            pltpu.VMEM((CHUNK,), jnp.int32),
            pltpu.VMEM((CHUNK, D), _f32),
            pltpu.VMEM_SHARED((NPAD, D), _f32),
            pltpu.SemaphoreType.DMA,
        ],
    )
    def propagate(g_hbm, rows_hbm, cols_hbm, zeros_hbm, acc_hbm,
                  idx_v, cidx_v, buf_v, acc_sh, sem):
        cid = lax.axis_index("c")
        sid = lax.axis_index("s")
        pltpu.sync_copy(zeros_hbm,
                        acc_sh.at[pl.ds(sid * ROWS_PER_TILE_PAD,
                                        ROWS_PER_TILE_PAD)])
        plsc.subcore_barrier()

        def body(c, carry):
            pltpu.sync_copy(rows_hbm.at[cid, sid, c], idx_v)
            pltpu.async_copy(g_hbm.at[idx_v], buf_v, sem).wait()
            pltpu.sync_copy(cols_hbm.at[cid, sid, c], cidx_v)
            pltpu.sync_copy(buf_v, acc_sh.at[cidx_v], add=True)
            return carry

        lax.fori_loop(0, CPT, body, 0)
        plsc.subcore_barrier()
        pltpu.sync_copy(
            acc_sh.at[pl.ds(sid * ROWS_PER_TILE_PAD, ROWS_PER_TILE_PAD)],
            acc_hbm.at[cid, pl.ds(sid * ROWS_PER_TILE_PAD,
                                  ROWS_PER_TILE_PAD)])

    return propagate


def _make_degree():
    @functools.partial(
        pl.kernel,
        mesh=_sc_mesh(),
        out_type=jax.ShapeDtypeStruct((NC, NPAD, 16), _f32),
        scratch_types=[
            pltpu.VMEM((CHUNK,), jnp.int32),
            pltpu.VMEM((CHUNK, 16), _f32),
            pltpu.VMEM_SHARED((NPAD, 16), _f32),
        ],
    )
    def degree(cols_hbm, ones_hbm, zeros_hbm, deg_hbm,
               cidx_v, ones_v, deg_sh):
        cid = lax.axis_index("c")
        sid = lax.axis_index("s")
        pltpu.sync_copy(ones_hbm, ones_v)
        pltpu.sync_copy(zeros_hbm,
                        deg_sh.at[pl.ds(sid * ROWS_PER_TILE_PAD,
                                        ROWS_PER_TILE_PAD)])
        plsc.subcore_barrier()

        def body(c, carry):
            pltpu.sync_copy(cols_hbm.at[cid, sid, c], cidx_v)
            pltpu.sync_copy(ones_v, deg_sh.at[cidx_v], add=True)
            return carry

        lax.fori_loop(0, CPT, body, 0)
        plsc.subcore_barrier()
        pltpu.sync_copy(
            deg_sh.at[pl.ds(sid * ROWS_PER_TILE_PAD, ROWS_PER_TILE_PAD)],
            deg_hbm.at[cid, pl.ds(sid * ROWS_PER_TILE_PAD,
                                  ROWS_PER_TILE_PAD)])

    return degree


def _mlp_call(x, w1t, b1r, w2t, b2r, degp):
    def body(x_ref, w1_ref, b1_ref, w2_ref, b2_ref, dg_ref, h0_ref, g0_ref):
        h1 = jnp.dot(x_ref[...], w1_ref[...],
                     preferred_element_type=_f32) + b1_ref[...]
        h1 = jnp.maximum(h1, 0.0)
        h2 = jnp.dot(h1, w2_ref[...],
                     preferred_element_type=_f32) + b2_ref[...]
        deg = dg_ref[0, :, 0:1] + dg_ref[1, :, 0:1]
        h0_ref[...] = h2
        g0_ref[...] = h2 * lax.rsqrt(deg)

    return pl.pallas_call(
        body,
        grid=(N // RB,),
        in_specs=[
            pl.BlockSpec((RB, D), lambda i: (i, 0)),
            pl.BlockSpec((D, D), lambda i: (0, 0)),
            pl.BlockSpec((1, D), lambda i: (0, 0)),
            pl.BlockSpec((D, D), lambda i: (0, 0)),
            pl.BlockSpec((1, D), lambda i: (0, 0)),
            pl.BlockSpec((NC, RB, 16), lambda i: (0, i, 0)),
        ],
        out_specs=[pl.BlockSpec((RB, D), lambda i: (i, 0)),
                   pl.BlockSpec((RB, D), lambda i: (i, 0))],
        out_shape=[jax.ShapeDtypeStruct((N, D), _f32),
                   jax.ShapeDtypeStruct((N, D), _f32)],
    )(x, w1t, b1r, w2t, b2r, degp)


def _update_call(acc, degp, g0):
    def body(acc_ref, dg_ref, g0_ref, g_ref):
        deg = dg_ref[0, :, 0:1] + dg_ref[1, :, 0:1]
        s = acc_ref[0] + acc_ref[1]
        g_ref[...] = ((1.0 - ALPHA) / deg) * s + ALPHA * g0_ref[...]

    return pl.pallas_call(
        body,
        grid=(N // RB,),
        in_specs=[
            pl.BlockSpec((NC, RB, D), lambda i: (0, i, 0)),
            pl.BlockSpec((NC, RB, 16), lambda i: (0, i, 0)),
            pl.BlockSpec((RB, D), lambda i: (i, 0)),
        ],
        out_specs=pl.BlockSpec((RB, D), lambda i: (i, 0)),
        out_shape=jax.ShapeDtypeStruct((N, D), _f32),
    )(acc, degp, g0)


def _final_call(acc, degp, h0):
    def body(acc_ref, dg_ref, h0_ref, o_ref):
        deg = dg_ref[0, :, 0:1] + dg_ref[1, :, 0:1]
        s = acc_ref[0] + acc_ref[1]
        h = (1.0 - ALPHA) * lax.rsqrt(deg) * s + ALPHA * h0_ref[...]
        m = jnp.max(h, axis=1, keepdims=True)
        ex = jnp.exp(h - m)
        o_ref[...] = h - m - jnp.log(jnp.sum(ex, axis=1, keepdims=True))

    return pl.pallas_call(
        body,
        grid=(N // RB,),
        in_specs=[
            pl.BlockSpec((NC, RB, D), lambda i: (0, i, 0)),
            pl.BlockSpec((NC, RB, 16), lambda i: (0, i, 0)),
            pl.BlockSpec((RB, D), lambda i: (i, 0)),
        ],
        out_specs=pl.BlockSpec((RB, D), lambda i: (i, 0)),
        out_shape=jax.ShapeDtypeStruct((N, D), _f32),
    )(acc, degp, h0)


def kernel(x, edge_index, W1, b1, W2, b2):
    row = edge_index[0]
    col = edge_index[1]
    loop = jnp.arange(N, dtype=jnp.int32)
    rows = jnp.concatenate([row, loop,
                            jnp.zeros((CAP - ET,), jnp.int32)])
    cols = jnp.concatenate([col, loop,
                            jnp.full((CAP - ET,), N, jnp.int32)])
    rows_a = rows.reshape(NC, NS, CPT, CHUNK)
    cols_a = cols.reshape(NC, NS, CPT, CHUNK)

    zeros_d = jnp.zeros((ROWS_PER_TILE_PAD, D), _f32)
    zeros16 = jnp.zeros((ROWS_PER_TILE_PAD, 16), _f32)
    ones16 = jnp.ones((CHUNK, 16), _f32)

    degree = _make_degree()
    propagate = _make_propagate()

    degp = degree(cols_a, ones16, zeros16)
    h0, g0 = _mlp_call(x, W1.T, b1.reshape(1, D), W2.T, b2.reshape(1, D),
                       degp)

    g = g0
    for _ in range(K - 1):
        acc = propagate(g, rows_a, cols_a, zeros_d)
        g = _update_call(acc, degp, g0)
    acc = propagate(g, rows_a, cols_a, zeros_d)
    return _final_call(acc, degp, h0)


# speedup vs baseline: 2.2981x; 1.1153x over previous
import functools

import jax
import jax.numpy as jnp
from jax import lax
from jax.experimental import pallas as pl
from jax.experimental.pallas import tpu as pltpu
from jax.experimental.pallas import tpu_sc as plsc

N = 10000
NPAD = 10240
D = 128
E = 320000
ET = E + N
NC = 2
NS = 16
CHUNK = 128
CPT = 81
CAP = NC * NS * CPT * CHUNK
ROWS_PER_TILE_PAD = NPAD // NS
ALPHA = 0.1
K = 10
RB = 2000

_f32 = jnp.float32


def _sc_mesh():
    return plsc.VectorSubcoreMesh(core_axis_name="c", subcore_axis_name="s")


def _make_propagate():
    @functools.partial(
        pl.kernel,
        mesh=_sc_mesh(),
        out_type=jax.ShapeDtypeStruct((NC, NPAD, D), _f32),
        scratch_types=[
            pltpu.VMEM((CHUNK,), jnp.int32),
            pltpu.VMEM((CHUNK,), jnp.int32),
            pltpu.VMEM((CHUNK, D), _f32),
            pltpu.VMEM_SHARED((NPAD, D), _f32),
            pltpu.SemaphoreType.DMA,
        ],
    )
    def propagate(g_hbm, rows_hbm, cols_hbm, zeros_hbm, acc_hbm,
                  idx_v, cidx_v, buf_v, acc_sh, sem):
        cid = lax.axis_index("c")
        sid = lax.axis_index("s")
        pltpu.sync_copy(zeros_hbm,
                        acc_sh.at[pl.ds(sid * ROWS_PER_TILE_PAD,
                                        ROWS_PER_TILE_PAD)])
        plsc.subcore_barrier()

        def body(c, carry):
            pltpu.sync_copy(rows_hbm.at[cid, sid, c], idx_v)
            h = pltpu.async_copy(g_hbm.at[idx_v], buf_v, sem)
            pltpu.sync_copy(cols_hbm.at[cid, sid, c], cidx_v)
            h.wait()
            pltpu.sync_copy(buf_v, acc_sh.at[cidx_v], add=True)
            return carry

        lax.fori_loop(0, CPT, body, 0)
        plsc.subcore_barrier()
        pltpu.sync_copy(
            acc_sh.at[pl.ds(sid * ROWS_PER_TILE_PAD, ROWS_PER_TILE_PAD)],
            acc_hbm.at[cid, pl.ds(sid * ROWS_PER_TILE_PAD,
                                  ROWS_PER_TILE_PAD)])

    return propagate


def _make_degree():
    @functools.partial(
        pl.kernel,
        mesh=_sc_mesh(),
        out_type=jax.ShapeDtypeStruct((NC, NPAD, 16), _f32),
        scratch_types=[
            pltpu.VMEM((CHUNK,), jnp.int32),
            pltpu.VMEM((CHUNK, 16), _f32),
            pltpu.VMEM_SHARED((NPAD, 16), _f32),
        ],
    )
    def degree(cols_hbm, ones_hbm, zeros_hbm, deg_hbm,
               cidx_v, ones_v, deg_sh):
        cid = lax.axis_index("c")
        sid = lax.axis_index("s")
        pltpu.sync_copy(ones_hbm, ones_v)
        pltpu.sync_copy(zeros_hbm,
                        deg_sh.at[pl.ds(sid * ROWS_PER_TILE_PAD,
                                        ROWS_PER_TILE_PAD)])
        plsc.subcore_barrier()

        def body(c, carry):
            pltpu.sync_copy(cols_hbm.at[cid, sid, c], cidx_v)
            pltpu.sync_copy(ones_v, deg_sh.at[cidx_v], add=True)
            return carry

        lax.fori_loop(0, CPT, body, 0)
        plsc.subcore_barrier()
        pltpu.sync_copy(
            deg_sh.at[pl.ds(sid * ROWS_PER_TILE_PAD, ROWS_PER_TILE_PAD)],
            deg_hbm.at[cid, pl.ds(sid * ROWS_PER_TILE_PAD,
                                  ROWS_PER_TILE_PAD)])

    return degree


def _mlp_call(x, w1t, b1r, w2t, b2r, degp):
    def body(x_ref, w1_ref, b1_ref, w2_ref, b2_ref, dg_ref, h0_ref, g0_ref):
        h1 = jnp.dot(x_ref[...], w1_ref[...],
                     preferred_element_type=_f32) + b1_ref[...]
        h1 = jnp.maximum(h1, 0.0)
        h2 = jnp.dot(h1, w2_ref[...],
                     preferred_element_type=_f32) + b2_ref[...]
        deg = dg_ref[0, :, 0:1] + dg_ref[1, :, 0:1]
        h0_ref[...] = h2
        g0_ref[...] = h2 * lax.rsqrt(deg)

    return pl.pallas_call(
        body,
        grid=(N // RB,),
        in_specs=[
            pl.BlockSpec((RB, D), lambda i: (i, 0)),
            pl.BlockSpec((D, D), lambda i: (0, 0)),
            pl.BlockSpec((1, D), lambda i: (0, 0)),
            pl.BlockSpec((D, D), lambda i: (0, 0)),
            pl.BlockSpec((1, D), lambda i: (0, 0)),
            pl.BlockSpec((NC, RB, 16), lambda i: (0, i, 0)),
        ],
        out_specs=[pl.BlockSpec((RB, D), lambda i: (i, 0)),
                   pl.BlockSpec((RB, D), lambda i: (i, 0))],
        out_shape=[jax.ShapeDtypeStruct((N, D), _f32),
                   jax.ShapeDtypeStruct((N, D), _f32)],
    )(x, w1t, b1r, w2t, b2r, degp)


def _update_call(acc, degp, g0):
    def body(acc_ref, dg_ref, g0_ref, g_ref):
        deg = dg_ref[0, :, 0:1] + dg_ref[1, :, 0:1]
        s = acc_ref[0] + acc_ref[1]
        g_ref[...] = ((1.0 - ALPHA) / deg) * s + ALPHA * g0_ref[...]

    return pl.pallas_call(
        body,
        grid=(N // RB,),
        in_specs=[
            pl.BlockSpec((NC, RB, D), lambda i: (0, i, 0)),
            pl.BlockSpec((NC, RB, 16), lambda i: (0, i, 0)),
            pl.BlockSpec((RB, D), lambda i: (i, 0)),
        ],
        out_specs=pl.BlockSpec((RB, D), lambda i: (i, 0)),
        out_shape=jax.ShapeDtypeStruct((N, D), _f32),
    )(acc, degp, g0)


def _final_call(acc, degp, h0):
    def body(acc_ref, dg_ref, h0_ref, o_ref):
        deg = dg_ref[0, :, 0:1] + dg_ref[1, :, 0:1]
        s = acc_ref[0] + acc_ref[1]
        h = (1.0 - ALPHA) * lax.rsqrt(deg) * s + ALPHA * h0_ref[...]
        m = jnp.max(h, axis=1, keepdims=True)
        ex = jnp.exp(h - m)
        o_ref[...] = h - m - jnp.log(jnp.sum(ex, axis=1, keepdims=True))

    return pl.pallas_call(
        body,
        grid=(N // RB,),
        in_specs=[
            pl.BlockSpec((NC, RB, D), lambda i: (0, i, 0)),
            pl.BlockSpec((NC, RB, 16), lambda i: (0, i, 0)),
            pl.BlockSpec((RB, D), lambda i: (i, 0)),
        ],
        out_specs=pl.BlockSpec((RB, D), lambda i: (i, 0)),
        out_shape=jax.ShapeDtypeStruct((N, D), _f32),
    )(acc, degp, h0)


def kernel(x, edge_index, W1, b1, W2, b2):
    row = edge_index[0]
    col = edge_index[1]
    loop = jnp.arange(N, dtype=jnp.int32)
    rows = jnp.concatenate([row, loop,
                            jnp.zeros((CAP - ET,), jnp.int32)])
    cols = jnp.concatenate([col, loop,
                            jnp.full((CAP - ET,), N, jnp.int32)])
    rows_a = rows.reshape(NC, NS, CPT, CHUNK)
    cols_a = cols.reshape(NC, NS, CPT, CHUNK)

    zeros_d = jnp.zeros((ROWS_PER_TILE_PAD, D), _f32)
    zeros16 = jnp.zeros((ROWS_PER_TILE_PAD, 16), _f32)
    ones16 = jnp.ones((CHUNK, 16), _f32)

    degree = _make_degree()
    propagate = _make_propagate()

    degp = degree(cols_a, ones16, zeros16)
    h0, g0 = _mlp_call(x, W1.T, b1.reshape(1, D), W2.T, b2.reshape(1, D),
                       degp)

    g = g0
    for _ in range(K - 1):
        acc = propagate(g, rows_a, cols_a, zeros_d)
        g = _update_call(acc, degp, g0)
    acc = propagate(g, rows_a, cols_a, zeros_d)
    return _final_call(acc, degp, h0)
